# baseline (speedup 1.0000x reference)
import functools

import jax
import jax.numpy as jnp
from jax import lax
from jax.experimental import pallas as pl
from jax.experimental.pallas import tpu as pltpu
from jax.experimental.pallas import tpu_sc as plsc

IN_DIM = 4096
OUT_DIM = 8192
BATCH = 4096

NC = 2
NS = 16
NW = NC * NS
L = 16

ROWS_PER_TILE = BATCH // NW
R = 4
CW = 512


def _coef_body(w_cols):
    m = w_cols[0]
    for g in range(1, 16):
        m = jnp.maximum(m, w_cols[g])
    e = [jnp.exp(w_cols[g] - m) for g in range(16)]
    s = e[0]
    for g in range(1, 16):
        s = s + e[g]
    inv = 1.0 / s
    ca = (e[2] + e[3]) + (e[6] + e[7]) - (e[8] + e[9]) - (e[12] + e[13])
    cb = (e[4] + e[5]) + (e[6] + e[7]) - (e[8] + e[9]) - (e[10] + e[11])
    cab = (e[1] - e[2]) - (e[4] + e[7]) - 2.0 * (e[6] - e[9]) \
        + (e[8] + e[11]) + (e[13] - e[14])
    c0 = (e[8] + e[9]) + (e[10] + e[11]) + (e[12] + e[13]) + (e[14] + e[15])
    return c0 * inv, ca * inv, cb * inv, cab * inv


def _tec_body(x_hbm, wt_hbm, ia_hbm, ib_hbm, out_hbm,
              ia_v, ib_v, c0_v, ca_v, cb_v, cab_v, wbuf_v, xblk_v, obuf_v):
    cid = lax.axis_index("c")
    sid = lax.axis_index("s")
    wid = sid * NC + cid

    pltpu.sync_copy(ia_hbm, ia_v)
    pltpu.sync_copy(ib_hbm, ib_v)

    for c in range(OUT_DIM // CW):
        pltpu.sync_copy(wt_hbm.at[:, pl.ds(c * CW, CW)], wbuf_v)

        def coef_group(j, _, c=c):
            w_cols = [wbuf_v[g, pl.ds(j * L, L)] for g in range(16)]
            c0, ca, cb, cab = _coef_body(w_cols)
            base = c * CW + j * L
            c0_v[pl.ds(base, L)] = c0
            ca_v[pl.ds(base, L)] = ca
            cb_v[pl.ds(base, L)] = cb
            cab_v[pl.ds(base, L)] = cab
            return _

        lax.fori_loop(0, CW // L, coef_group, 0)

    row0 = wid * ROWS_PER_TILE

    def row_block(blk, _):
        rbase = row0 + blk * R
        pltpu.sync_copy(x_hbm.at[pl.ds(rbase, R)], xblk_v)

        def ochunk(o, _):
            ob = o * L
            ia = ia_v[pl.ds(ob, L)]
            ib = ib_v[pl.ds(ob, L)]
            c0 = c0_v[pl.ds(ob, L)]
            ca = ca_v[pl.ds(ob, L)]
            cb = cb_v[pl.ds(ob, L)]
            cab = cab_v[pl.ds(ob, L)]
            for r in range(R):
                rv = jnp.full((L,), r, dtype=jnp.int32)
                a = plsc.load_gather(xblk_v, [rv, ia])
                b = plsc.load_gather(xblk_v, [rv, ib])
                obuf_v[r, pl.ds(ob, L)] = (c0 + ca * a) + (cb + cab * a) * b
            return _

        lax.fori_loop(0, OUT_DIM // L, ochunk, 0)
        pltpu.sync_copy(obuf_v, out_hbm.at[pl.ds(rbase, R)])
        return _

    lax.fori_loop(0, ROWS_PER_TILE // R, row_block, 0)


@jax.jit
def _logic_layer_sc(x, wt, idx_a, idx_b):
    mesh = plsc.VectorSubcoreMesh(core_axis_name="c", subcore_axis_name="s")
    f = functools.partial(
        pl.kernel,
        mesh=mesh,
        compiler_params=pltpu.CompilerParams(needs_layout_passes=False),
        out_type=jax.ShapeDtypeStruct((BATCH, OUT_DIM), jnp.float32),
        scratch_types=[
            pltpu.VMEM((OUT_DIM,), jnp.int32),
            pltpu.VMEM((OUT_DIM,), jnp.int32),
            pltpu.VMEM((OUT_DIM,), jnp.float32),
            pltpu.VMEM((OUT_DIM,), jnp.float32),
            pltpu.VMEM((OUT_DIM,), jnp.float32),
            pltpu.VMEM((OUT_DIM,), jnp.float32),
            pltpu.VMEM((16, CW), jnp.float32),
            pltpu.VMEM((R, IN_DIM), jnp.float32),
            pltpu.VMEM((R, OUT_DIM), jnp.float32),
        ],
    )(_tec_body)
    return f(x, wt, idx_a, idx_b)


def kernel(x, weights, idx_a, idx_b):
    wt = jnp.transpose(weights)
    return _logic_layer_sc(x, wt, idx_a, idx_b)


# speedup vs baseline: 3.2083x; 3.2083x over previous
import functools

import jax
import jax.numpy as jnp
from jax import lax
from jax.experimental import pallas as pl
from jax.experimental.pallas import tpu as pltpu
from jax.experimental.pallas import tpu_sc as plsc

IN_DIM = 4096
OUT_DIM = 8192
BATCH = 4096

NC = 2
NS = 16
NW = NC * NS
L = 16

ROWS_PER_TILE = BATCH // NW
R = 8
NBLK = ROWS_PER_TILE // R
S = 512
NSTRIP = OUT_DIM // S
CPS = S // L
CW = 128


def _coef_body(w_cols):
    m = w_cols[0]
    for g in range(1, 16):
        m = jnp.maximum(m, w_cols[g])
    e = [jnp.exp(w_cols[g] - m) for g in range(16)]
    s = e[0]
    for g in range(1, 16):
        s = s + e[g]
    inv = 1.0 / s
    ca = (e[2] + e[3]) + (e[6] + e[7]) - (e[8] + e[9]) - (e[12] + e[13])
    cb = (e[4] + e[5]) + (e[6] + e[7]) - (e[8] + e[9]) - (e[10] + e[11])
    cab = (e[1] - e[2]) - (e[4] + e[7]) - 2.0 * (e[6] - e[9]) \
        + (e[8] + e[11]) + (e[13] - e[14])
    c0 = (e[8] + e[9]) + (e[10] + e[11]) + (e[12] + e[13]) + (e[14] + e[15])
    return c0 * inv, ca * inv, cb * inv, cab * inv


def _tec_body(x_hbm, wt_hbm, ia_hbm, ib_hbm, out_hbm,
              ia_v, ib_v, c0_v, ca_v, cb_v, cab_v, wbuf_v, xblk_v, obuf_v,
              sem_x, sem_o):
    cid = lax.axis_index("c")
    sid = lax.axis_index("s")
    wid = sid * NC + cid

    pltpu.sync_copy(ia_hbm, ia_v)
    pltpu.sync_copy(ib_hbm, ib_v)

    def coef_chunk(c, carry):
        pltpu.sync_copy(wt_hbm.at[:, pl.ds(c * CW, CW)], wbuf_v)

        def coef_group(j, carry2):
            w_cols = [wbuf_v[g, pl.ds(j * L, L)] for g in range(16)]
            c0, ca, cb, cab = _coef_body(w_cols)
            base = c * CW + j * L
            c0_v[pl.ds(base, L)] = c0
            ca_v[pl.ds(base, L)] = ca
            cb_v[pl.ds(base, L)] = cb
            cab_v[pl.ds(base, L)] = cab
            return carry2

        lax.fori_loop(0, CW // L, coef_group, 0)
        return carry

    lax.fori_loop(0, OUT_DIM // CW, coef_chunk, 0)

    row0 = wid * ROWS_PER_TILE

    pltpu.async_copy(x_hbm.at[pl.ds(row0, R)], xblk_v.at[0], sem_x)

    def row_block(blk, carry):
        px = blk & 1
        rbase = row0 + blk * R
        pltpu.make_async_copy(
            x_hbm.at[pl.ds(rbase, R)], xblk_v.at[px], sem_x).wait()

        @pl.when(blk + 1 < NBLK)
        def _():
            pltpu.async_copy(
                x_hbm.at[pl.ds(rbase + R, R)], xblk_v.at[1 - px], sem_x)

        pxv = jnp.broadcast_to(px, (L,)).astype(jnp.int32)

        def strip(st, carry2):
            pb = st & 1
            gst = blk * NSTRIP + st
            obase = st * S

            @pl.when(gst >= 2)
            def _():
                pltpu.make_async_copy(
                    obuf_v.at[pb],
                    out_hbm.at[pl.ds(rbase, R), pl.ds(obase, S)],
                    sem_o).wait()

            @plsc.parallel_loop(0, CPS, 1, unroll=2)
            def chunk(oc):
                o16 = obase + oc * L
                ia = ia_v[pl.ds(o16, L)]
                ib = ib_v[pl.ds(o16, L)]
                c0 = c0_v[pl.ds(o16, L)]
                ca = ca_v[pl.ds(o16, L)]
                cb = cb_v[pl.ds(o16, L)]
                cab = cab_v[pl.ds(o16, L)]
                for r in range(R):
                    rv = jnp.full((L,), r, dtype=jnp.int32)
                    a = plsc.load_gather(xblk_v, [pxv, rv, ia])
                    b = plsc.load_gather(xblk_v, [pxv, rv, ib])
                    obuf_v[pb, r, pl.ds(oc * L, L)] = \
                        (c0 + ca * a) + (cb + cab * a) * b

            pltpu.async_copy(
                obuf_v.at[pb],
                out_hbm.at[pl.ds(rbase, R), pl.ds(obase, S)],
                sem_o)
            return carry2

        lax.fori_loop(0, NSTRIP, strip, 0)
        return carry

    lax.fori_loop(0, NBLK, row_block, 0)

    for _ in range(2):
        pltpu.make_async_copy(
            obuf_v.at[0],
            out_hbm.at[pl.ds(row0, R), pl.ds(0, S)],
            sem_o).wait()


@jax.jit
def _logic_layer_sc(x, wt, idx_a, idx_b):
    mesh = plsc.VectorSubcoreMesh(core_axis_name="c", subcore_axis_name="s")
    f = functools.partial(
        pl.kernel,
        mesh=mesh,
        compiler_params=pltpu.CompilerParams(needs_layout_passes=False),
        out_type=jax.ShapeDtypeStruct((BATCH, OUT_DIM), jnp.float32),
        scratch_types=[
            pltpu.VMEM((OUT_DIM,), jnp.int32),
            pltpu.VMEM((OUT_DIM,), jnp.int32),
            pltpu.VMEM((OUT_DIM,), jnp.float32),
            pltpu.VMEM((OUT_DIM,), jnp.float32),
            pltpu.VMEM((OUT_DIM,), jnp.float32),
            pltpu.VMEM((OUT_DIM,), jnp.float32),
            pltpu.VMEM((16, CW), jnp.float32),
            pltpu.VMEM((2, R, IN_DIM), jnp.float32),
            pltpu.VMEM((2, R, S), jnp.float32),
            pltpu.SemaphoreType.DMA,
            pltpu.SemaphoreType.DMA,
        ],
    )(_tec_body)
    return f(x, wt, idx_a, idx_b)


def kernel(x, weights, idx_a, idx_b):
    wt = jnp.transpose(weights)
    return _logic_layer_sc(x, wt, idx_a, idx_b)


# speedup vs baseline: 3.7328x; 1.1635x over previous
import functools

import jax
import jax.numpy as jnp
from jax import lax
from jax.experimental import pallas as pl
from jax.experimental.pallas import tpu as pltpu
from jax.experimental.pallas import tpu_sc as plsc

IN_DIM = 4096
OUT_DIM = 8192
BATCH = 4096

NC = 2
NS = 16
NW = NC * NS
L = 16

ROWS_PER_TILE = BATCH // NW
R = 8
NBLK = ROWS_PER_TILE // R
S = 1024
NSTRIP = OUT_DIM // S
CPS = S // L
CW = 128
NCW = OUT_DIM // CW


def _round_bf16_bits(v):
    bits = plsc.bitcast(v, jnp.uint32)
    return lax.shift_right_logical(bits + jnp.uint32(0x8000),
                                   jnp.uint32(16))


def _coef_body(w_cols):
    m = w_cols[0]
    for g in range(1, 16):
        m = jnp.maximum(m, w_cols[g])
    e = [jnp.exp(w_cols[g] - m) for g in range(16)]
    s = e[0]
    for g in range(1, 16):
        s = s + e[g]
    inv = 1.0 / s
    ca = (e[2] + e[3]) + (e[6] + e[7]) - (e[8] + e[9]) - (e[12] + e[13])
    cb = (e[4] + e[5]) + (e[6] + e[7]) - (e[8] + e[9]) - (e[10] + e[11])
    cab = (e[1] - e[2]) - (e[4] + e[7]) - 2.0 * (e[6] - e[9]) \
        + (e[8] + e[11]) + (e[13] - e[14])
    c0 = (e[8] + e[9]) + (e[10] + e[11]) + (e[12] + e[13]) + (e[14] + e[15])
    return c0 * inv, ca * inv, cb * inv, cab * inv


def _unpack_pair(pk):
    lo = plsc.bitcast(lax.shift_left(pk, jnp.uint32(16)), jnp.float32)
    hi = plsc.bitcast(pk & jnp.uint32(0xFFFF0000), jnp.float32)
    return lo, hi


def _tec_body(x_hbm, wt_hbm, ia_hbm, ib_hbm, out_hbm,
              iab_v, stage_v, cp0_v, cp1_v, wbuf_v, xblk_v, obuf_v,
              sem_x, sem_o):
    cid = lax.axis_index("c")
    sid = lax.axis_index("s")
    wid = sid * NC + cid

    pltpu.sync_copy(ia_hbm, iab_v)
    pltpu.sync_copy(ib_hbm, stage_v)

    def pack_idx(j, carry):
        ds = pl.ds(j * L, L)
        iab_v[ds] = iab_v[ds] | lax.shift_left(stage_v[ds], 16)
        return carry

    lax.fori_loop(0, OUT_DIM // L, pack_idx, 0)

    pltpu.async_copy(wt_hbm.at[:, pl.ds(0, CW)], wbuf_v.at[0], sem_x)

    def coef_chunk(c, carry):
        pw = c & 1
        pltpu.make_async_copy(
            wt_hbm.at[:, pl.ds(c * CW, CW)], wbuf_v.at[pw], sem_x).wait()

        @pl.when(c + 1 < NCW)
        def _():
            pltpu.async_copy(
                wt_hbm.at[:, pl.ds((c + 1) * CW, CW)], wbuf_v.at[1 - pw],
                sem_x)

        def coef_group(j, carry2):
            w_cols = [wbuf_v[pw, g, pl.ds(j * L, L)] for g in range(16)]
            c0, ca, cb, cab = _coef_body(w_cols)
            base = pl.ds(c * CW + j * L, L)
            cp0_v[base] = _round_bf16_bits(c0) | \
                lax.shift_left(_round_bf16_bits(ca), jnp.uint32(16))
            cp1_v[base] = _round_bf16_bits(cb) | \
                lax.shift_left(_round_bf16_bits(cab), jnp.uint32(16))
            return carry2

        lax.fori_loop(0, CW // L, coef_group, 0)
        return carry

    lax.fori_loop(0, NCW, coef_chunk, 0)

    row0 = wid * ROWS_PER_TILE

    pltpu.async_copy(x_hbm.at[pl.ds(row0, R)], xblk_v.at[0], sem_x)

    def row_block(blk, carry):
        px = blk & 1
        rbase = row0 + blk * R
        pltpu.make_async_copy(
            x_hbm.at[pl.ds(rbase, R)], xblk_v.at[px], sem_x).wait()

        @pl.when(blk + 1 < NBLK)
        def _():
            pltpu.async_copy(
                x_hbm.at[pl.ds(rbase + R, R)], xblk_v.at[1 - px], sem_x)

        pxv = jnp.broadcast_to(px, (L,)).astype(jnp.int32)

        def strip(st, carry2):
            pb = st & 1
            gst = blk * NSTRIP + st
            obase = st * S

            @pl.when(gst >= 2)
            def _():
                pltpu.make_async_copy(
                    obuf_v.at[pb],
                    out_hbm.at[pl.ds(rbase, R), pl.ds(obase, S)],
                    sem_o).wait()

            @plsc.parallel_loop(0, CPS, 1, unroll=2)
            def chunk(oc):
                ds = pl.ds(obase + oc * L, L)
                pk = iab_v[ds]
                ia = pk & jnp.int32(0xFFFF)
                ib = lax.shift_right_logical(pk, 16)
                c0, ca = _unpack_pair(cp0_v[ds])
                cb, cab = _unpack_pair(cp1_v[ds])
                for r in range(R):
                    rv = jnp.full((L,), r, dtype=jnp.int32)
                    a = plsc.load_gather(xblk_v, [pxv, rv, ia])
                    b = plsc.load_gather(xblk_v, [pxv, rv, ib])
                    obuf_v[pb, r, pl.ds(oc * L, L)] = \
                        (c0 + ca * a) + (cb + cab * a) * b

            pltpu.async_copy(
                obuf_v.at[pb],
                out_hbm.at[pl.ds(rbase, R), pl.ds(obase, S)],
                sem_o)
            return carry2

        lax.fori_loop(0, NSTRIP, strip, 0)
        return carry

    lax.fori_loop(0, NBLK, row_block, 0)

    for _ in range(2):
        pltpu.make_async_copy(
            obuf_v.at[0],
            out_hbm.at[pl.ds(row0, R), pl.ds(0, S)],
            sem_o).wait()


@jax.jit
def _logic_layer_sc(x, wt, idx_a, idx_b):
    mesh = plsc.VectorSubcoreMesh(core_axis_name="c", subcore_axis_name="s")
    f = functools.partial(
        pl.kernel,
        mesh=mesh,
        compiler_params=pltpu.CompilerParams(needs_layout_passes=False),
        out_type=jax.ShapeDtypeStruct((BATCH, OUT_DIM), jnp.float32),
        scratch_types=[
            pltpu.VMEM((OUT_DIM,), jnp.int32),
            pltpu.VMEM((OUT_DIM,), jnp.int32),
            pltpu.VMEM((OUT_DIM,), jnp.uint32),
            pltpu.VMEM((OUT_DIM,), jnp.uint32),
            pltpu.VMEM((2, 16, CW), jnp.float32),
            pltpu.VMEM((2, R, IN_DIM), jnp.float32),
            pltpu.VMEM((2, R, S), jnp.float32),
            pltpu.SemaphoreType.DMA,
            pltpu.SemaphoreType.DMA,
        ],
    )(_tec_body)
    return f(x, wt, idx_a, idx_b)


def kernel(x, weights, idx_a, idx_b):
    wt = jnp.transpose(weights)
    return _logic_layer_sc(x, wt, idx_a, idx_b)
